# baseline (speedup 1.0000x reference)
import functools

import jax
import jax.numpy as jnp
from jax import lax
from jax.experimental import pallas as pl
from jax.experimental.pallas import tpu as pltpu
from jax.experimental.pallas import tpu_sc as plsc

D = 32
CHUNK = 128


def _gather_sc(table, idx3, n_chunks):
    NW = idx3.shape[0]
    NC = 2
    per_w = n_chunks * CHUNK
    R = NW * per_w
    mesh = plsc.VectorSubcoreMesh(core_axis_name="c", subcore_axis_name="s")

    @functools.partial(
        pl.kernel,
        out_type=jax.ShapeDtypeStruct((R, D), jnp.float32),
        mesh=mesh,
        compiler_params=pltpu.CompilerParams(use_tc_tiling_on_sc=False),
        scratch_types=[
            pltpu.VMEM((n_chunks, CHUNK), jnp.int32),
            pltpu.VMEM((per_w, D), jnp.float32),
            pltpu.SemaphoreType.DMA,
        ],
    )
    def k(table_hbm, idx_hbm, out_hbm, idx_v, rows_v, sem):
        wid = lax.axis_index("s") * NC + lax.axis_index("c")
        base = wid * per_w
        pltpu.sync_copy(idx_hbm.at[wid], idx_v)
        handles = []
        for c in range(n_chunks):
            handles.append(
                pltpu.async_copy(
                    table_hbm.at[idx_v.at[c]],
                    rows_v.at[pl.ds(c * CHUNK, CHUNK)],
                    sem,
                )
            )
        for h in handles:
            h.wait()
        pltpu.sync_copy(rows_v, out_hbm.at[pl.ds(base, per_w)])

    return k(table, idx3)


def _cosine_tc(rows, batch):
    def body(r_ref, o_ref):
        r = r_ref[...]
        e1 = r[:, :D]
        e2 = r[:, D:]
        dot = jnp.sum(e1 * e2, axis=1)
        s1 = jnp.sum(e1 * e1, axis=1)
        s2 = jnp.sum(e2 * e2, axis=1)
        eps = jnp.float32(1e-8)
        n1 = jnp.maximum(jnp.sqrt(s1), eps)
        n2 = jnp.maximum(jnp.sqrt(s2), eps)
        o_ref[...] = 0.5 + 0.5 * (dot / (n1 * n2))

    return pl.pallas_call(
        body,
        out_shape=jax.ShapeDtypeStruct((batch,), jnp.float32),
    )(rows)


def kernel(x, table):
    x = x.reshape(-1, 2)
    batch = x.shape[0]
    idx_flat = x.reshape(-1).astype(jnp.int32)
    NW = 32
    n_chunks = (2 * batch) // (NW * CHUNK)
    idx3 = idx_flat.reshape(NW, n_chunks, CHUNK)
    rows = _gather_sc(table, idx3, n_chunks)
    rows2 = rows.reshape(batch, 2 * D)
    return _cosine_tc(rows2, batch)
